# baseline (speedup 1.0000x reference)
import functools

import jax
import jax.numpy as jnp
from jax import lax
from jax.experimental import pallas as pl
from jax.experimental.pallas import tpu as pltpu
from jax.experimental.pallas import tpu_sc as plsc

VOCAB = 10
HIDDEN = 128
NC, NS = 2, 16
NW = NC * NS
CHUNK = 128
NBUF = 6
HBM_SLOTS = (2, 5)
NCOPIES = 128


@functools.partial(jax.jit, static_argnames=("nidx",))
def _emb_lookup(idx, table, table_rep, nidx):
    nchunks = nidx

    @functools.partial(
        pl.kernel,
        out_type=jax.ShapeDtypeStruct((NW * nidx * CHUNK, HIDDEN), jnp.float32),
        mesh=plsc.VectorSubcoreMesh(core_axis_name="c", subcore_axis_name="s"),
        scratch_types=[
            pltpu.VMEM((nidx, CHUNK), jnp.int32),
            pltpu.VMEM((NBUF, CHUNK, HIDDEN), jnp.float32),
            pltpu.VMEM_SHARED((NS * VOCAB, HIDDEN), jnp.float32),
            [pltpu.SemaphoreType.DMA] * NBUF,
            [pltpu.SemaphoreType.DMA] * NBUF,
        ],
    )
    def k(idx_hbm, table_hbm, trep_hbm, out_hbm, idx_v, rbuf, table_sp, gs, ws):
        wid = lax.axis_index("s") * NC + lax.axis_index("c")

        @pl.when(lax.axis_index("s") == 0)
        def _():
            pltpu.sync_copy(table_hbm, table_sp)

        pltpu.sync_copy(idx_hbm.at[wid], idx_v)
        plsc.subcore_barrier()

        def start_gather(j, b):
            src = trep_hbm if b in HBM_SLOTS else table_sp
            pltpu.async_copy(src.at[idx_v.at[j]], rbuf.at[b], gs[b])

        def wait_gather(b):
            pltpu.make_async_copy(table_sp, rbuf.at[b], gs[b]).wait()

        def out_slice(j):
            return out_hbm.at[pl.ds((wid * nchunks + j) * CHUNK, CHUNK)]

        def wait_write(j, b):
            pltpu.make_async_copy(rbuf.at[b], out_slice(j), ws[b]).wait()

        for j in range(4):
            start_gather(j, j)

        def step(j, b):
            p = (b + 4) % NBUF
            wait_gather(b)
            pltpu.async_copy(rbuf.at[b], out_slice(j), ws[b])

            @pl.when(j + 4 < nchunks)
            def _():
                @pl.when(j >= 2)
                def _():
                    wait_write(j - 2, p)

                start_gather(j + 4, p)

        nmain = (nchunks // NBUF) * NBUF

        def body(i, carry):
            for u in range(NBUF):
                step(i * NBUF + u, u)
            return carry

        lax.fori_loop(0, nmain // NBUF, body, 0)
        for j in range(nmain, nchunks):
            step(j, j % NBUF)
        for j in range(nchunks - NBUF, nchunks):
            wait_write(j, j % NBUF)

    return k(idx, table, table_rep)


def kernel(input_ids, word_embeddings):
    b, s = input_ids.shape
    n = b * s
    assert n % (NW * CHUNK) == 0
    nidx = n // (NW * CHUNK)
    idx = input_ids.reshape(NW, nidx, CHUNK).astype(jnp.int32)
    sp_idx = idx + ((jnp.arange(NW, dtype=jnp.int32) // NC) * VOCAB)[:, None, None]
    hbm_idx = idx * NCOPIES + jnp.arange(CHUNK, dtype=jnp.int32)
    is_hbm = (jnp.arange(nidx, dtype=jnp.int32) % 3) == 2
    idx = jnp.where(is_hbm[None, :, None], hbm_idx, sp_idx)
    table_tiled = jnp.tile(word_embeddings, (NS, 1))
    table_rep = jnp.repeat(word_embeddings, NCOPIES, axis=0)
    out = _emb_lookup(idx, table_tiled, table_rep, nidx)
    return out.reshape(b, s, HIDDEN)


# speedup vs baseline: 1.3739x; 1.3739x over previous
import functools

import jax
import jax.numpy as jnp
from jax import lax
from jax.experimental import pallas as pl
from jax.experimental.pallas import tpu as pltpu
from jax.experimental.pallas import tpu_sc as plsc

VOCAB = 10
HIDDEN = 128
NC, NS = 2, 16
NW = NC * NS
CHUNK = 128
NBUF = 6


@functools.partial(jax.jit, static_argnames=("nidx",))
def _emb_lookup(idx, table, nidx):
    nchunks = nidx

    @functools.partial(
        pl.kernel,
        out_type=jax.ShapeDtypeStruct((NW * nidx * CHUNK, HIDDEN), jnp.float32),
        mesh=plsc.VectorSubcoreMesh(core_axis_name="c", subcore_axis_name="s"),
        scratch_types=[
            pltpu.VMEM((nidx, CHUNK), jnp.int32),
            pltpu.VMEM((NBUF, CHUNK, HIDDEN), jnp.float32),
            pltpu.VMEM_SHARED((NS * VOCAB, HIDDEN), jnp.float32),
            [pltpu.SemaphoreType.DMA] * NBUF,
            [pltpu.SemaphoreType.DMA] * NBUF,
        ],
    )
    def k(idx_hbm, table_hbm, out_hbm, idx_v, rbuf, table_sp, gs, ws):
        wid = lax.axis_index("s") * NC + lax.axis_index("c")

        @pl.when(lax.axis_index("s") == 0)
        def _():
            pltpu.sync_copy(table_hbm, table_sp)

        pltpu.sync_copy(idx_hbm.at[wid], idx_v)
        plsc.subcore_barrier()

        def start_gather(j, b):
            pltpu.async_copy(table_sp.at[idx_v.at[j]], rbuf.at[b], gs[b])

        def wait_gather(b):
            pltpu.make_async_copy(table_sp, rbuf.at[b], gs[b]).wait()

        def out_slice(j):
            return out_hbm.at[pl.ds((wid * nchunks + j) * CHUNK, CHUNK)]

        def wait_write(j, b):
            pltpu.make_async_copy(rbuf.at[b], out_slice(j), ws[b]).wait()

        for j in range(4):
            start_gather(j, j)

        def step(j, b):
            p = (b + 4) % NBUF
            wait_gather(b)

            @pl.when(j + 4 < nchunks)
            def _():
                @pl.when(j >= 2)
                def _():
                    wait_write(j - 2, p)

                start_gather(j + 4, p)

            pltpu.async_copy(rbuf.at[b], out_slice(j), ws[b])

        nmain = (nchunks // NBUF) * NBUF

        def body(i, carry):
            for u in range(NBUF):
                step(i * NBUF + u, u)
            return carry

        lax.fori_loop(0, nmain // NBUF, body, 0)
        for j in range(nmain, nchunks):
            step(j, j % NBUF)
        for j in range(nchunks - NBUF, nchunks):
            wait_write(j, j % NBUF)

    return k(idx, table)


def kernel(input_ids, word_embeddings):
    b, s = input_ids.shape
    n = b * s
    assert n % (NW * CHUNK) == 0
    nidx = n // (NW * CHUNK)
    idx = input_ids.reshape(NW, nidx, CHUNK).astype(jnp.int32)
    idx = idx + ((jnp.arange(NW, dtype=jnp.int32) // NC) * VOCAB)[:, None, None]
    table_tiled = jnp.tile(word_embeddings, (NS, 1))
    out = _emb_lookup(idx, table_tiled, nidx)
    return out.reshape(b, s, HIDDEN)
